# initial kernel scaffold (unmeasured)
import jax
import jax.numpy as jnp
from jax import lax
from jax.experimental import pallas as pl
from jax.experimental.pallas import tpu as pltpu

N_Y = 4
SC = 512
EB = 2048
N_EB = 4
N_STEP = N_Y - 1


def kernel(O, Wo):
    B, S, Hs, D = O.shape
    K = Hs * D
    E = Wo.shape[1]

    O16 = O.reshape(B, S, K).astype(jnp.bfloat16)
    W16 = Wo.astype(jnp.bfloat16)
    P16 = jnp.matmul(O16, W16)

    def body(p_ref, out_ref, send_ref, comm_ref, local_ref, outstage_ref,
             send_sems, recv_sems, credit_sems, local_sem):
        my_x = lax.axis_index("x")
        my_y = lax.axis_index("y")
        my_z = lax.axis_index("z")
        right = (my_x, (my_y + 1) % N_Y, my_z)
        left = (my_x, (my_y - 1) % N_Y, my_z)

        def p_slice(chunk, eb):
            return p_ref.at[:, pl.ds(chunk * SC, SC), pl.ds(eb * EB, EB)]

        rdmas = {}
        for k in range(N_EB * N_STEP):
            eb, t = divmod(k, N_STEP)
            slot = k % 2
            c_send = (my_y - 1 - t) % N_Y
            c_recv = (my_y - 2 - t) % N_Y

            if k >= 2:
                rdmas[k - 2].wait_send()

            if t == 0:
                cp = pltpu.make_async_copy(
                    p_slice(c_send, eb), send_ref.at[slot], local_sem)
                cp.start()
                cp.wait()
            else:
                prev = (k - 1) % 2
                send_ref[slot] = comm_ref[prev] + local_ref[...]
                pl.semaphore_signal(
                    credit_sems.at[prev], inc=1,
                    device_id=left, device_id_type=pl.DeviceIdType.MESH)

            if k >= 2:
                pl.semaphore_wait(credit_sems.at[slot], 1)

            rd = pltpu.make_async_remote_copy(
                src_ref=send_ref.at[slot],
                dst_ref=comm_ref.at[slot],
                send_sem=send_sems.at[slot],
                recv_sem=recv_sems.at[slot],
                device_id=right,
                device_id_type=pl.DeviceIdType.MESH,
            )
            rd.start()
            rdmas[k] = rd

            cp = pltpu.make_async_copy(
                p_slice(c_recv, eb), local_ref, local_sem)
            cp.start()
            cp.wait()

            rd.wait_recv()

            if t == N_STEP - 1:
                outstage_ref[...] = (comm_ref[slot].astype(jnp.float32)
                                     + local_ref[...].astype(jnp.float32))
                pl.semaphore_signal(
                    credit_sems.at[slot], inc=1,
                    device_id=left, device_id_type=pl.DeviceIdType.MESH)
                cpo = pltpu.make_async_copy(
                    outstage_ref,
                    out_ref.at[:, :, pl.ds(eb * EB, EB)],
                    local_sem)
                cpo.start()
                cpo.wait()

        last = N_EB * N_STEP
        rdmas[last - 2].wait_send()
        rdmas[last - 1].wait_send()
        pl.semaphore_wait(credit_sems.at[0], 1)
        pl.semaphore_wait(credit_sems.at[1], 1)

    return pl.pallas_call(
        body,
        out_shape=jax.ShapeDtypeStruct((B, SC, E), jnp.float32),
        in_specs=[pl.BlockSpec(memory_space=pltpu.ANY)],
        out_specs=pl.BlockSpec(memory_space=pltpu.ANY),
        scratch_shapes=[
            pltpu.VMEM((2, B, SC, EB), jnp.bfloat16),
            pltpu.VMEM((2, B, SC, EB), jnp.bfloat16),
            pltpu.VMEM((B, SC, EB), jnp.bfloat16),
            pltpu.VMEM((B, SC, EB), jnp.float32),
            pltpu.SemaphoreType.DMA((2,)),
            pltpu.SemaphoreType.DMA((2,)),
            pltpu.SemaphoreType.REGULAR((2,)),
            pltpu.SemaphoreType.DMA,
        ],
        compiler_params=pltpu.CompilerParams(collective_id=0),
    )(P16)


# baseline (device time: 1621549 ns/iter reference)
import jax
import jax.numpy as jnp
from jax import lax
from jax.experimental import pallas as pl
from jax.experimental.pallas import tpu as pltpu

N_Y = 4
SC = 512
EB = 2048
N_EB = 4
N_STEP = N_Y - 1


def kernel(O, Wo):
    B, S, Hs, D = O.shape
    K = Hs * D
    E = Wo.shape[1]

    O16 = O.reshape(B, S, K).astype(jnp.bfloat16)
    W16 = Wo.astype(jnp.bfloat16)
    P16 = jnp.matmul(O16, W16)

    def body(p_ref, out_ref, send_ref, comm_ref, local_ref, outstage_ref,
             send_sems, recv_sems, credit_sems, local_sem):
        my_x = lax.axis_index("x")
        my_y = lax.axis_index("y")
        my_z = lax.axis_index("z")
        right = (my_x, (my_y + 1) % N_Y, my_z)
        left = (my_x, (my_y - 1) % N_Y, my_z)

        def p_slice(chunk, eb):
            return p_ref.at[:, pl.ds(chunk * SC, SC), pl.ds(eb * EB, EB)]

        rdmas = {}
        for k in range(N_EB * N_STEP):
            eb, t = divmod(k, N_STEP)
            slot = k % 2
            c_send = (my_y - 1 - t) % N_Y
            c_recv = (my_y - 2 - t) % N_Y

            if k >= 2:
                rdmas[k - 2].wait_send()

            if t == 0:
                cp = pltpu.make_async_copy(
                    p_slice(c_send, eb), send_ref.at[slot], local_sem)
                cp.start()
                cp.wait()
            else:
                prev = (k - 1) % 2
                send_ref[slot] = comm_ref[prev] + local_ref[...]
                pl.semaphore_signal(
                    credit_sems.at[prev], inc=1,
                    device_id=left, device_id_type=pl.DeviceIdType.MESH)

            if k >= 2:
                pl.semaphore_wait(credit_sems.at[slot], 1)

            rd = pltpu.make_async_remote_copy(
                src_ref=send_ref.at[slot],
                dst_ref=comm_ref.at[slot],
                send_sem=send_sems.at[slot],
                recv_sem=recv_sems.at[slot],
                device_id=right,
                device_id_type=pl.DeviceIdType.MESH,
            )
            rd.start()
            rdmas[k] = rd

            cp = pltpu.make_async_copy(
                p_slice(c_recv, eb), local_ref, local_sem)
            cp.start()
            cp.wait()

            rd.wait_recv()

            if t == N_STEP - 1:
                outstage_ref[...] = (comm_ref[slot].astype(jnp.float32)
                                     + local_ref[...].astype(jnp.float32))
                pl.semaphore_signal(
                    credit_sems.at[slot], inc=1,
                    device_id=left, device_id_type=pl.DeviceIdType.MESH)
                cpo = pltpu.make_async_copy(
                    outstage_ref,
                    out_ref.at[:, :, pl.ds(eb * EB, EB)],
                    local_sem)
                cpo.start()
                cpo.wait()

        last = N_EB * N_STEP
        rdmas[last - 2].wait_send()
        rdmas[last - 1].wait_send()
        pl.semaphore_wait(credit_sems.at[0], 1)
        pl.semaphore_wait(credit_sems.at[1], 1)

    return pl.pallas_call(
        body,
        out_shape=jax.ShapeDtypeStruct((B, SC, E), jnp.float32),
        in_specs=[pl.BlockSpec(memory_space=pl.ANY)],
        out_specs=pl.BlockSpec(memory_space=pl.ANY),
        scratch_shapes=[
            pltpu.VMEM((2, B, SC, EB), jnp.bfloat16),
            pltpu.VMEM((2, B, SC, EB), jnp.bfloat16),
            pltpu.VMEM((B, SC, EB), jnp.bfloat16),
            pltpu.VMEM((B, SC, EB), jnp.float32),
            pltpu.SemaphoreType.DMA((2,)),
            pltpu.SemaphoreType.DMA((2,)),
            pltpu.SemaphoreType.REGULAR((2,)),
            pltpu.SemaphoreType.DMA,
        ],
        compiler_params=pltpu.CompilerParams(
            vmem_limit_bytes=64 * 1024 * 1024,
        ),
    )(P16)


# device time: 1455554 ns/iter; 1.1140x vs baseline; 1.1140x over previous
import jax
import jax.numpy as jnp
from jax import lax
from jax.experimental import pallas as pl
from jax.experimental.pallas import tpu as pltpu

N_Y = 4
SC = 512
EB = 1024
N_EB = 8
N_STEP = N_Y - 1
R = 2048


def kernel(O, Wo):
    B, S, Hs, D = O.shape
    K = Hs * D
    E = Wo.shape[1]

    O16 = O.reshape(B, S, K).astype(jnp.bfloat16)
    Oc = O16.reshape(B, N_Y, SC, K).swapaxes(0, 1).reshape(N_Y, B * SC, K)
    W16 = Wo.astype(jnp.bfloat16)

    def body(oc_ref, w_ref, out_ref, send_ref, comm_ref, pout_ref,
             ostage_ref, wstage_ref,
             send_sems, recv_sems, credit_sems, o_sem, w_sem):
        my_x = lax.axis_index("x")
        my_y = lax.axis_index("y")
        my_z = lax.axis_index("z")
        right = (my_x, (my_y + 1) % N_Y, my_z)
        left = (my_x, (my_y - 1) % N_Y, my_z)

        def signal_credit(slot):
            pl.semaphore_signal(
                credit_sems.at[slot], inc=1,
                device_id=left, device_id_type=pl.DeviceIdType.MESH)

        def ring_rdma(slot):
            return pltpu.make_async_remote_copy(
                src_ref=send_ref.at[slot],
                dst_ref=comm_ref.at[slot],
                send_sem=send_sems.at[slot],
                recv_sem=recv_sems.at[slot],
                device_id=right,
                device_id_type=pl.DeviceIdType.MESH,
            )

        def local_partial(chunk, eb):
            co = pltpu.make_async_copy(oc_ref.at[chunk], ostage_ref, o_sem)
            co.start()
            co.wait()
            return jnp.dot(
                ostage_ref[...], wstage_ref[...],
                preferred_element_type=jnp.float32,
            ).astype(jnp.bfloat16)

        def eb_body(eb, carry):
            ecols = pl.ds(eb * EB, EB)

            cw = pltpu.make_async_copy(w_ref.at[:, ecols], wstage_ref, w_sem)
            cw.start()
            cw.wait()

            send_ref[0] = local_partial((my_y - 1) % N_Y, eb)

            @pl.when(eb > 0)
            def _():
                pl.semaphore_wait(credit_sems.at[0], 1)
            rd0 = ring_rdma(0)
            rd0.start()

            pout_ref[...] = local_partial((my_y - 2) % N_Y, eb)
            rd0.wait_recv()

            send_ref[1] = comm_ref[0] + pout_ref[...]
            signal_credit(0)

            @pl.when(eb > 0)
            def _():
                pl.semaphore_wait(credit_sems.at[1], 1)
            rd1 = ring_rdma(1)
            rd1.start()

            pout_ref[...] = local_partial((my_y - 3) % N_Y, eb)
            rd1.wait_recv()

            rd0.wait_send()
            send_ref[0] = comm_ref[1] + pout_ref[...]
            signal_credit(1)

            pl.semaphore_wait(credit_sems.at[0], 1)
            rd2 = ring_rdma(0)
            rd2.start()

            pout_ref[...] = local_partial(my_y, eb)
            rd2.wait_recv()

            pout_ref[...] = comm_ref[0] + pout_ref[...]
            signal_credit(0)
            cpo = pltpu.make_async_copy(pout_ref, out_ref.at[:, ecols], o_sem)
            cpo.start()
            cpo.wait()

            rd1.wait_send()
            rd2.wait_send()
            return carry

        lax.fori_loop(0, N_EB, eb_body, 0)

        pl.semaphore_wait(credit_sems.at[0], 1)
        pl.semaphore_wait(credit_sems.at[1], 1)

    flat = pl.pallas_call(
        body,
        out_shape=jax.ShapeDtypeStruct((R, E), jnp.bfloat16),
        in_specs=[pl.BlockSpec(memory_space=pl.ANY),
                  pl.BlockSpec(memory_space=pl.ANY)],
        out_specs=pl.BlockSpec(memory_space=pl.ANY),
        scratch_shapes=[
            pltpu.VMEM((2, R, EB), jnp.bfloat16),
            pltpu.VMEM((2, R, EB), jnp.bfloat16),
            pltpu.VMEM((R, EB), jnp.bfloat16),
            pltpu.VMEM((R, K), jnp.bfloat16),
            pltpu.VMEM((K, EB), jnp.bfloat16),
            pltpu.SemaphoreType.DMA((2,)),
            pltpu.SemaphoreType.DMA((2,)),
            pltpu.SemaphoreType.REGULAR((2,)),
            pltpu.SemaphoreType.DMA,
            pltpu.SemaphoreType.DMA,
        ],
        compiler_params=pltpu.CompilerParams(
            vmem_limit_bytes=64 * 1024 * 1024,
        ),
    )(Oc, W16)
    return flat.astype(jnp.float32).reshape(B, SC, E)


# device time: 1350884 ns/iter; 1.2004x vs baseline; 1.0775x over previous
import jax
import jax.numpy as jnp
from jax import lax
from jax.experimental import pallas as pl
from jax.experimental.pallas import tpu as pltpu

N_Y = 4
SC = 512
EB = 1024
N_EB = 8
R = 2048
RH = R // 2


def kernel(O, Wo):
    B, S, Hs, D = O.shape
    K = Hs * D
    E = Wo.shape[1]

    O16 = O.reshape(B, S, K).astype(jnp.bfloat16)
    Oc = O16.reshape(B, N_Y, SC, K).swapaxes(0, 1).reshape(N_Y, B * SC, K)
    W16 = Wo.astype(jnp.bfloat16)

    def body(oc_ref, w_ref, out_ref, send_ref, comm_ref,
             ostage_ref, wstage_ref,
             send_sems, recv_sems, credit_sems, o_sem, w_sem):
        my_x = lax.axis_index("x")
        my_y = lax.axis_index("y")
        my_z = lax.axis_index("z")
        right = (my_x, (my_y + 1) % N_Y, my_z)
        left = (my_x, (my_y - 1) % N_Y, my_z)

        def signal_credit(slot):
            pl.semaphore_signal(
                credit_sems.at[slot], inc=1,
                device_id=left, device_id_type=pl.DeviceIdType.MESH)

        def ring_rdma(sslot, cslot):
            return pltpu.make_async_remote_copy(
                src_ref=send_ref.at[sslot],
                dst_ref=comm_ref.at[cslot],
                send_sem=send_sems.at[sslot],
                recv_sem=recv_sems.at[cslot],
                device_id=right,
                device_id_type=pl.DeviceIdType.MESH,
            )

        def load_w(eb):
            cw = pltpu.make_async_copy(
                w_ref.at[:, pl.ds(eb * EB, EB)], wstage_ref, w_sem)
            cw.start()
            cw.wait()

        def partial_into_send(sslot, chunk):
            co = pltpu.make_async_copy(oc_ref.at[chunk], ostage_ref, o_sem)
            co.start()
            co.wait()
            for h in range(2):
                rows = pl.ds(h * RH, RH)
                send_ref[sslot, rows] = jnp.dot(
                    ostage_ref[rows, :], wstage_ref[...],
                    preferred_element_type=jnp.float32,
                ).astype(jnp.bfloat16)

        def accumulate_send(sslot, cslot):
            for h in range(2):
                rows = pl.ds(h * RH, RH)
                send_ref[sslot, rows] = (send_ref[sslot, rows]
                                         + comm_ref[cslot, rows])

        load_w(0)
        partial_into_send(0, (my_y - 1) % N_Y)

        def eb_body(eb, carry):
            ecols = pl.ds(eb * EB, EB)

            @pl.when(eb > 0)
            def _():
                pl.semaphore_wait(credit_sems.at[0], 1)
            rd0 = ring_rdma(0, 0)
            rd0.start()

            partial_into_send(1, (my_y - 2) % N_Y)
            rd0.wait_recv()

            accumulate_send(1, 0)
            signal_credit(0)

            @pl.when(eb > 0)
            def _():
                pl.semaphore_wait(credit_sems.at[1], 1)
            rd1 = ring_rdma(1, 1)
            rd1.start()

            @pl.when(eb > 0)
            def _():
                ring_rdma(2, 0).wait_send()
            partial_into_send(2, (my_y - 3) % N_Y)
            rd1.wait_recv()

            accumulate_send(2, 1)
            signal_credit(1)

            pl.semaphore_wait(credit_sems.at[0], 1)
            rd2 = ring_rdma(2, 0)
            rd2.start()

            rd1.wait_send()
            partial_into_send(1, my_y)

            rd0.wait_send()

            @pl.when(eb < N_EB - 1)
            def _():
                cw = pltpu.make_async_copy(
                    w_ref.at[:, pl.ds((eb + 1) * EB, EB)], wstage_ref, w_sem)
                cw.start()
                cw.wait()
                partial_into_send(0, (my_y - 1) % N_Y)

            rd2.wait_recv()

            accumulate_send(1, 0)
            signal_credit(0)
            cpo = pltpu.make_async_copy(
                send_ref.at[1], out_ref.at[:, ecols], o_sem)
            cpo.start()
            cpo.wait()
            return carry

        lax.fori_loop(0, N_EB, eb_body, 0)

        ring_rdma(2, 0).wait_send()
        pl.semaphore_wait(credit_sems.at[0], 1)
        pl.semaphore_wait(credit_sems.at[1], 1)

    flat = pl.pallas_call(
        body,
        out_shape=jax.ShapeDtypeStruct((R, E), jnp.bfloat16),
        in_specs=[pl.BlockSpec(memory_space=pl.ANY),
                  pl.BlockSpec(memory_space=pl.ANY)],
        out_specs=pl.BlockSpec(memory_space=pl.ANY),
        scratch_shapes=[
            pltpu.VMEM((3, R, EB), jnp.bfloat16),
            pltpu.VMEM((2, R, EB), jnp.bfloat16),
            pltpu.VMEM((R, K), jnp.bfloat16),
            pltpu.VMEM((K, EB), jnp.bfloat16),
            pltpu.SemaphoreType.DMA((3,)),
            pltpu.SemaphoreType.DMA((2,)),
            pltpu.SemaphoreType.REGULAR((2,)),
            pltpu.SemaphoreType.DMA,
            pltpu.SemaphoreType.DMA,
        ],
        compiler_params=pltpu.CompilerParams(
            vmem_limit_bytes=64 * 1024 * 1024,
        ),
    )(Oc, W16)
    return flat.astype(jnp.float32).reshape(B, SC, E)


# device time: 1324240 ns/iter; 1.2245x vs baseline; 1.0201x over previous
import jax
import jax.numpy as jnp
from jax import lax
from jax.experimental import pallas as pl
from jax.experimental.pallas import tpu as pltpu

N_Y = 4
SC = 512
EB = 1024
N_EB = 8
R = 2048
RH = R // 2


def kernel(O, Wo):
    B, S, Hs, D = O.shape
    K = Hs * D
    E = Wo.shape[1]

    O16 = O.reshape(B, S, K).astype(jnp.bfloat16)
    Oc = O16.reshape(B, N_Y, SC, K).swapaxes(0, 1).reshape(N_Y, B * SC, K)

    def body(oc_ref, w_ref, out_ref, send_ref, comm_ref,
             ostage_ref, wstage_ref, wstage32_ref,
             send_sems, recv_sems, credit_sems, o_sem, w_sem):
        my_x = lax.axis_index("x")
        my_y = lax.axis_index("y")
        my_z = lax.axis_index("z")
        right = (my_x, (my_y + 1) % N_Y, my_z)
        left = (my_x, (my_y - 1) % N_Y, my_z)

        def signal_credit(slot):
            pl.semaphore_signal(
                credit_sems.at[slot], inc=1,
                device_id=left, device_id_type=pl.DeviceIdType.MESH)

        def ring_rdma(sslot, cslot):
            return pltpu.make_async_remote_copy(
                src_ref=send_ref.at[sslot],
                dst_ref=comm_ref.at[cslot],
                send_sem=send_sems.at[sslot],
                recv_sem=recv_sems.at[cslot],
                device_id=right,
                device_id_type=pl.DeviceIdType.MESH,
            )

        def load_w(eb):
            cw = pltpu.make_async_copy(
                w_ref.at[:, pl.ds(eb * EB, EB)], wstage32_ref, w_sem)
            cw.start()
            cw.wait()
            for h in range(2):
                rows = pl.ds(h * RH, RH)
                wstage_ref[rows] = wstage32_ref[rows].astype(jnp.bfloat16)

        def partial_into_send(sslot, chunk):
            co = pltpu.make_async_copy(oc_ref.at[chunk], ostage_ref, o_sem)
            co.start()
            co.wait()
            for h in range(2):
                rows = pl.ds(h * RH, RH)
                send_ref[sslot, rows] = jnp.dot(
                    ostage_ref[rows, :], wstage_ref[...],
                    preferred_element_type=jnp.float32,
                ).astype(jnp.bfloat16)

        def accumulate_send(sslot, cslot):
            for h in range(2):
                rows = pl.ds(h * RH, RH)
                send_ref[sslot, rows] = (send_ref[sslot, rows]
                                         + comm_ref[cslot, rows])

        load_w(0)
        partial_into_send(0, (my_y - 1) % N_Y)

        def eb_body(eb, carry):
            ecols = pl.ds(eb * EB, EB)

            @pl.when(eb > 0)
            def _():
                pl.semaphore_wait(credit_sems.at[0], 1)
            rd0 = ring_rdma(0, 0)
            rd0.start()

            partial_into_send(1, (my_y - 2) % N_Y)
            rd0.wait_recv()

            accumulate_send(1, 0)
            signal_credit(0)

            @pl.when(eb > 0)
            def _():
                pl.semaphore_wait(credit_sems.at[1], 1)
            rd1 = ring_rdma(1, 1)
            rd1.start()

            @pl.when(eb > 0)
            def _():
                ring_rdma(2, 0).wait_send()
            partial_into_send(2, (my_y - 3) % N_Y)
            rd1.wait_recv()

            accumulate_send(2, 1)
            signal_credit(1)

            pl.semaphore_wait(credit_sems.at[0], 1)
            rd2 = ring_rdma(2, 0)
            rd2.start()

            rd1.wait_send()
            partial_into_send(1, my_y)

            rd0.wait_send()

            @pl.when(eb < N_EB - 1)
            def _():
                load_w(eb + 1)
                partial_into_send(0, (my_y - 1) % N_Y)

            rd2.wait_recv()

            accumulate_send(1, 0)
            signal_credit(0)
            cpo = pltpu.make_async_copy(
                send_ref.at[1], out_ref.at[:, ecols], o_sem)
            cpo.start()
            cpo.wait()
            return carry

        lax.fori_loop(0, N_EB, eb_body, 0)

        ring_rdma(2, 0).wait_send()
        pl.semaphore_wait(credit_sems.at[0], 1)
        pl.semaphore_wait(credit_sems.at[1], 1)

    flat = pl.pallas_call(
        body,
        out_shape=jax.ShapeDtypeStruct((R, E), jnp.bfloat16),
        in_specs=[pl.BlockSpec(memory_space=pl.ANY),
                  pl.BlockSpec(memory_space=pl.ANY)],
        out_specs=pl.BlockSpec(memory_space=pl.ANY),
        scratch_shapes=[
            pltpu.VMEM((3, R, EB), jnp.bfloat16),
            pltpu.VMEM((2, R, EB), jnp.bfloat16),
            pltpu.VMEM((R, K), jnp.bfloat16),
            pltpu.VMEM((K, EB), jnp.bfloat16),
            pltpu.VMEM((K, EB), jnp.float32),
            pltpu.SemaphoreType.DMA((3,)),
            pltpu.SemaphoreType.DMA((2,)),
            pltpu.SemaphoreType.REGULAR((2,)),
            pltpu.SemaphoreType.DMA,
            pltpu.SemaphoreType.DMA,
        ],
        compiler_params=pltpu.CompilerParams(
            vmem_limit_bytes=64 * 1024 * 1024,
        ),
    )(Oc, Wo)
    return flat.astype(jnp.float32).reshape(B, SC, E)
